# baseline (speedup 1.0000x reference)
import functools

import jax
import jax.numpy as jnp
from jax.experimental import pallas as pl
from jax.experimental.pallas import tpu as pltpu
from jax.experimental.pallas import tpu_sc as plsc

TOKENS = 2048
D = 1024
F = 2048
E = 8
K = 2
TILE = 128
NT = (TOKENS * K) // TILE + E
R_PAD = NT * TILE
GW = 32
CW = 16

@functools.lru_cache(maxsize=1)
def _sc_mesh():
    return plsc.VectorSubcoreMesh(core_axis_name="c", subcore_axis_name="s")


def _dispatch_meta(expert_ids, expert_weights):
    N = TOKENS * K
    C = 128
    NCH = N // C

    flat_e = expert_ids.reshape(-1).astype(jnp.int32)
    flat_w = expert_weights.reshape(-1).astype(jnp.float32)

    ohf = (flat_e[:, None] == jnp.arange(E, dtype=jnp.int32)[None, :]).astype(
        jnp.float32
    )
    ohc = ohf.astype(jnp.bfloat16).reshape(NCH, C, E)
    tri = jnp.tril(jnp.ones((C, C), jnp.bfloat16))
    within = jnp.einsum(
        "rc,bce->bre", tri, ohc, preferred_element_type=jnp.float32
    )
    chunk_tot = within[:, C - 1, :]
    trix = jnp.tril(jnp.ones((NCH, NCH), jnp.bfloat16), k=-1)
    chunk_base = jnp.einsum(
        "bc,ce->be",
        trix,
        chunk_tot.astype(jnp.bfloat16),
        preferred_element_type=jnp.float32,
    )
    incl = (within + chunk_base[:, None, :]).reshape(N, E)
    rank = jnp.sum((incl - 1.0) * ohf, axis=1)

    counts = jnp.sum(chunk_tot, axis=0).astype(jnp.int32)
    tiles_per_e = (counts + TILE - 1) // TILE
    cum_tiles = jnp.cumsum(tiles_per_e)
    row_base = (cum_tiles - tiles_per_e) * TILE
    pos = (
        jnp.sum(row_base.astype(jnp.float32)[None, :] * ohf, axis=1) + rank
    ).astype(jnp.int32)

    w_pad = jnp.zeros((R_PAD,), jnp.float32).at[pos].set(flat_w)

    nvalid = cum_tiles[E - 1].astype(jnp.int32)
    tr = jnp.arange(NT, dtype=jnp.int32)
    te_raw = jnp.sum(
        (tr[:, None] >= cum_tiles[None, :]).astype(jnp.int32), axis=1
    )
    last_e = jnp.sum((cum_tiles <= nvalid - 1).astype(jnp.int32))
    te = jnp.where(tr < nvalid, jnp.minimum(te_raw, E - 1), last_e)

    p0 = pos[0::2].astype(jnp.int32)
    p1 = pos[1::2].astype(jnp.int32)
    return w_pad, te, nvalid.reshape(1), p0, p1


def _sc_dispatch(x, p0, p1):
    NW = 32
    t_per_w = TOKENS // NW

    @functools.partial(
        pl.kernel,
        out_type=jax.ShapeDtypeStruct((R_PAD, D), jnp.float32),
        mesh=_sc_mesh(),
        scratch_types=[
            pltpu.VMEM((t_per_w,), jnp.int32),
            pltpu.VMEM((t_per_w,), jnp.int32),
            pltpu.VMEM((t_per_w, D), jnp.float32),
            pltpu.SemaphoreType.DMA,
            pltpu.SemaphoreType.DMA,
        ],
    )
    def k(x_hbm, p0_hbm, p1_hbm, o_hbm, i0_v, i1_v, xrows, sem0, sem1):
        wid = jax.lax.axis_index("s") * 2 + jax.lax.axis_index("c")
        base = wid * t_per_w
        pltpu.sync_copy(p0_hbm.at[pl.ds(base, t_per_w)], i0_v)
        pltpu.sync_copy(p1_hbm.at[pl.ds(base, t_per_w)], i1_v)
        pltpu.sync_copy(x_hbm.at[pl.ds(base, t_per_w)], xrows)
        c0 = pltpu.async_copy(xrows, o_hbm.at[i0_v], sem0)
        c1 = pltpu.async_copy(xrows, o_hbm.at[i1_v], sem1)
        c0.wait()
        c1.wait()

    return k(x, p0, p1)


def _mlp_body(te_ref, nv_ref, xs_ref, ws_ref, g_ref, u_ref, d_ref, ys_ref):
    i = pl.program_id(0)

    @pl.when(i < nv_ref[0])
    def _():
        xb = xs_ref[...].astype(jnp.bfloat16)
        g = jnp.dot(xb, g_ref[0], preferred_element_type=jnp.float32)
        u = jnp.dot(xb, u_ref[0], preferred_element_type=jnp.float32)
        h = (g * jax.nn.sigmoid(g) * u).astype(jnp.bfloat16)
        o = jnp.dot(h, d_ref[0], preferred_element_type=jnp.float32)
        ys_ref[...] = o * ws_ref[...]


def _tc_grouped_mlp(xs, w_pad, te, nv, gw, uw, dw):
    grid_spec = pltpu.PrefetchScalarGridSpec(
        num_scalar_prefetch=2,
        grid=(NT,),
        in_specs=[
            pl.BlockSpec((TILE, D), lambda i, te, nv: (i, 0)),
            pl.BlockSpec((TILE, 1), lambda i, te, nv: (i, 0)),
            pl.BlockSpec((1, D, F), lambda i, te, nv: (te[i], 0, 0)),
            pl.BlockSpec((1, D, F), lambda i, te, nv: (te[i], 0, 0)),
            pl.BlockSpec((1, F, D), lambda i, te, nv: (te[i], 0, 0)),
        ],
        out_specs=pl.BlockSpec((TILE, D), lambda i, te, nv: (i, 0)),
    )
    return pl.pallas_call(
        _mlp_body,
        grid_spec=grid_spec,
        out_shape=jax.ShapeDtypeStruct((R_PAD, D), jnp.float32),
    )(te, nv, xs, w_pad.reshape(R_PAD, 1), gw, uw, dw)


def _sc_combine(ys, p0, p1):
    NW = 32
    t_per_w = TOKENS // NW

    @functools.partial(
        pl.kernel,
        out_type=jax.ShapeDtypeStruct((TOKENS, D), jnp.float32),
        mesh=_sc_mesh(),
        scratch_types=[
            pltpu.VMEM((t_per_w,), jnp.int32),
            pltpu.VMEM((t_per_w,), jnp.int32),
            pltpu.VMEM((CW, D), jnp.float32),
            pltpu.VMEM((CW, D), jnp.float32),
            pltpu.SemaphoreType.DMA,
            pltpu.SemaphoreType.DMA,
        ],
    )
    def k(ys_hbm, p0_hbm, p1_hbm, o_hbm, i0_v, i1_v, buf0, buf1, sem0, sem1):
        wid = jax.lax.axis_index("s") * 2 + jax.lax.axis_index("c")
        base = wid * t_per_w
        pltpu.sync_copy(p0_hbm.at[pl.ds(base, t_per_w)], i0_v)
        pltpu.sync_copy(p1_hbm.at[pl.ds(base, t_per_w)], i1_v)

        @pl.loop(0, t_per_w, step=CW)
        def _(c):
            cp0 = pltpu.async_copy(ys_hbm.at[i0_v.at[pl.ds(c, CW)]], buf0, sem0)
            cp1 = pltpu.async_copy(ys_hbm.at[i1_v.at[pl.ds(c, CW)]], buf1, sem1)
            cp0.wait()
            cp1.wait()

            @pl.loop(0, CW)
            def _(r):
                @pl.loop(0, D, step=64)
                def _(col):
                    for u in range(4):
                        slc = (pl.ds(r, 1), pl.ds(col + u * 16, 16))
                        buf0.at[slc[0], slc[1]][...] = (
                            buf0.at[slc[0], slc[1]][...]
                            + buf1.at[slc[0], slc[1]][...]
                        )

            pltpu.sync_copy(buf0, o_hbm.at[pl.ds(base + c, CW)])

    return k(ys, p0, p1)


def kernel(x, expert_ids, expert_weights, gate_weights, up_weights, down_weights):
    w_pad, te, nv, p0, p1 = _dispatch_meta(expert_ids, expert_weights)
    xs = _sc_dispatch(x, p0, p1)
    ys = _tc_grouped_mlp(xs, w_pad, te, nv, gate_weights.astype(jnp.bfloat16),
                         up_weights.astype(jnp.bfloat16), down_weights.astype(jnp.bfloat16))
    return _sc_combine(ys, p0, p1)


# speedup vs baseline: 1.3797x; 1.3797x over previous
import functools

import jax
import jax.numpy as jnp
from jax.experimental import pallas as pl
from jax.experimental.pallas import tpu as pltpu
from jax.experimental.pallas import tpu_sc as plsc

TOKENS = 2048
D = 1024
F = 2048
E = 8
K = 2
TILE = 128
NT = (TOKENS * K) // TILE + E
R_PAD = NT * TILE
GW = 32
CW = 16

@functools.lru_cache(maxsize=1)
def _sc_mesh():
    return plsc.VectorSubcoreMesh(core_axis_name="c", subcore_axis_name="s")


def _dispatch_meta(expert_ids, expert_weights):
    N = TOKENS * K
    C = 128
    NCH = N // C

    flat_e = expert_ids.reshape(-1).astype(jnp.int32)
    flat_w = expert_weights.reshape(-1).astype(jnp.float32)

    ohf = (flat_e[:, None] == jnp.arange(E, dtype=jnp.int32)[None, :]).astype(
        jnp.float32
    )
    ohc = ohf.astype(jnp.bfloat16).reshape(NCH, C, E)
    tri = jnp.tril(jnp.ones((C, C), jnp.bfloat16))
    within = jnp.einsum(
        "rc,bce->bre", tri, ohc, preferred_element_type=jnp.float32
    )
    chunk_tot = within[:, C - 1, :]
    trix = jnp.tril(jnp.ones((NCH, NCH), jnp.bfloat16), k=-1)
    chunk_base = jnp.einsum(
        "bc,ce->be",
        trix,
        chunk_tot.astype(jnp.bfloat16),
        preferred_element_type=jnp.float32,
    )
    incl = (within + chunk_base[:, None, :]).reshape(N, E)
    rank = jnp.sum((incl - 1.0) * ohf, axis=1)

    counts = jnp.sum(chunk_tot, axis=0).astype(jnp.int32)
    tiles_per_e = (counts + TILE - 1) // TILE
    cum_tiles = jnp.cumsum(tiles_per_e)
    row_base = (cum_tiles - tiles_per_e) * TILE
    pos = (
        jnp.sum(row_base.astype(jnp.float32)[None, :] * ohf, axis=1) + rank
    ).astype(jnp.int32)

    w_pad = jnp.zeros((R_PAD,), jnp.float32).at[pos].set(flat_w)

    nvalid = cum_tiles[E - 1].astype(jnp.int32)
    tr = jnp.arange(NT, dtype=jnp.int32)
    te_raw = jnp.sum(
        (tr[:, None] >= cum_tiles[None, :]).astype(jnp.int32), axis=1
    )
    last_e = jnp.sum((cum_tiles <= nvalid - 1).astype(jnp.int32))
    te = jnp.where(tr < nvalid, jnp.minimum(te_raw, E - 1), last_e)

    p0 = pos[0::2].astype(jnp.int32)
    p1 = pos[1::2].astype(jnp.int32)

    prev = jnp.concatenate([jnp.full((1,), -1, jnp.int32), te[:-1]])
    chg = (te != prev).astype(jnp.int32)
    first = jnp.where(tr < nvalid, chg, 0).astype(jnp.int32)
    slot = ((jnp.cumsum(chg) - 1) % 2).astype(jnp.int32)
    er = jnp.arange(E, dtype=jnp.int32)
    cand = jnp.where(
        (er[None, :] > er[:, None]) & (counts[None, :] > 0), er[None, :], E
    )
    npx = jnp.min(cand, axis=1)
    npx = jnp.where(npx == E, -1, npx)
    nxte = jnp.sum(
        jnp.where(te[:, None] == er[None, :], npx[None, :], 0), axis=1
    ).astype(jnp.int32)
    return w_pad, te, nvalid.reshape(1), p0, p1, slot, first, nxte


def _sc_dispatch(x, p0, p1):
    NW = 32
    t_per_w = TOKENS // NW

    @functools.partial(
        pl.kernel,
        out_type=jax.ShapeDtypeStruct((R_PAD, D), jnp.float32),
        mesh=_sc_mesh(),
        scratch_types=[
            pltpu.VMEM((t_per_w,), jnp.int32),
            pltpu.VMEM((t_per_w,), jnp.int32),
            pltpu.VMEM((t_per_w, D), jnp.float32),
            pltpu.SemaphoreType.DMA,
            pltpu.SemaphoreType.DMA,
        ],
    )
    def k(x_hbm, p0_hbm, p1_hbm, o_hbm, i0_v, i1_v, xrows, sem0, sem1):
        wid = jax.lax.axis_index("s") * 2 + jax.lax.axis_index("c")
        base = wid * t_per_w
        pltpu.sync_copy(p0_hbm.at[pl.ds(base, t_per_w)], i0_v)
        pltpu.sync_copy(p1_hbm.at[pl.ds(base, t_per_w)], i1_v)
        pltpu.sync_copy(x_hbm.at[pl.ds(base, t_per_w)], xrows)
        c0 = pltpu.async_copy(xrows, o_hbm.at[i0_v], sem0)
        c1 = pltpu.async_copy(xrows, o_hbm.at[i1_v], sem1)
        c0.wait()
        c1.wait()

    return k(x, p0, p1)


def _mlp_body(
    te_ref,
    nv_ref,
    slot_ref,
    first_ref,
    nxte_ref,
    xs_ref,
    ws_ref,
    g_hbm,
    u_hbm,
    d_hbm,
    ys_ref,
    gbuf,
    ubuf,
    dbuf,
    gsem,
    usem,
    dsem,
):
    i = pl.program_id(0)

    def fetch(e, s):
        pltpu.make_async_copy(g_hbm.at[e], gbuf.at[s], gsem.at[s]).start()
        pltpu.make_async_copy(u_hbm.at[e], ubuf.at[s], usem.at[s]).start()
        pltpu.make_async_copy(d_hbm.at[e], dbuf.at[s], dsem.at[s]).start()

    def wait(e, s):
        pltpu.make_async_copy(g_hbm.at[e], gbuf.at[s], gsem.at[s]).wait()
        pltpu.make_async_copy(u_hbm.at[e], ubuf.at[s], usem.at[s]).wait()
        pltpu.make_async_copy(d_hbm.at[e], dbuf.at[s], dsem.at[s]).wait()

    @pl.when(i == 0)
    def _():
        fetch(te_ref[0], 0)

    s = slot_ref[i]

    @pl.when(jnp.logical_and(first_ref[i] == 1, i < nv_ref[0]))
    def _():
        wait(te_ref[i], s)
        nxt = nxte_ref[i]

        @pl.when(nxt >= 0)
        def _():
            fetch(nxt, 1 - s)

    @pl.when(i < nv_ref[0])
    def _():
        xb = xs_ref[...].astype(jnp.bfloat16)
        gw = gbuf[s].astype(jnp.bfloat16)
        uw = ubuf[s].astype(jnp.bfloat16)
        dw = dbuf[s].astype(jnp.bfloat16)
        g = jnp.dot(xb, gw, preferred_element_type=jnp.float32)
        u = jnp.dot(xb, uw, preferred_element_type=jnp.float32)
        h = (g * jax.nn.sigmoid(g) * u).astype(jnp.bfloat16)
        o = jnp.dot(h, dw, preferred_element_type=jnp.float32)
        ys_ref[...] = o * ws_ref[...]


def _tc_grouped_mlp(xs, w_pad, te, nv, gw, uw, dw, slot, first, nxte):
    grid_spec = pltpu.PrefetchScalarGridSpec(
        num_scalar_prefetch=5,
        grid=(NT,),
        in_specs=[
            pl.BlockSpec((TILE, D), lambda i, *_: (i, 0)),
            pl.BlockSpec((TILE, 1), lambda i, *_: (i, 0)),
            pl.BlockSpec(memory_space=pl.ANY),
            pl.BlockSpec(memory_space=pl.ANY),
            pl.BlockSpec(memory_space=pl.ANY),
        ],
        out_specs=pl.BlockSpec((TILE, D), lambda i, *_: (i, 0)),
        scratch_shapes=[
            pltpu.VMEM((2, D, F), jnp.float32),
            pltpu.VMEM((2, D, F), jnp.float32),
            pltpu.VMEM((2, F, D), jnp.float32),
            pltpu.SemaphoreType.DMA((2,)),
            pltpu.SemaphoreType.DMA((2,)),
            pltpu.SemaphoreType.DMA((2,)),
        ],
    )
    return pl.pallas_call(
        _mlp_body,
        grid_spec=grid_spec,
        out_shape=jax.ShapeDtypeStruct((R_PAD, D), jnp.float32),
    )(te, nv, slot, first, nxte, xs, w_pad.reshape(R_PAD, 1), gw, uw, dw)


def _sc_combine(ys, p0, p1):
    NW = 32
    t_per_w = TOKENS // NW

    @functools.partial(
        pl.kernel,
        out_type=jax.ShapeDtypeStruct((TOKENS, D), jnp.float32),
        mesh=_sc_mesh(),
        scratch_types=[
            pltpu.VMEM((t_per_w,), jnp.int32),
            pltpu.VMEM((t_per_w,), jnp.int32),
            pltpu.VMEM((CW, D), jnp.float32),
            pltpu.VMEM((CW, D), jnp.float32),
            pltpu.SemaphoreType.DMA,
            pltpu.SemaphoreType.DMA,
        ],
    )
    def k(ys_hbm, p0_hbm, p1_hbm, o_hbm, i0_v, i1_v, buf0, buf1, sem0, sem1):
        wid = jax.lax.axis_index("s") * 2 + jax.lax.axis_index("c")
        base = wid * t_per_w
        pltpu.sync_copy(p0_hbm.at[pl.ds(base, t_per_w)], i0_v)
        pltpu.sync_copy(p1_hbm.at[pl.ds(base, t_per_w)], i1_v)

        @pl.loop(0, t_per_w, step=CW)
        def _(c):
            cp0 = pltpu.async_copy(ys_hbm.at[i0_v.at[pl.ds(c, CW)]], buf0, sem0)
            cp1 = pltpu.async_copy(ys_hbm.at[i1_v.at[pl.ds(c, CW)]], buf1, sem1)
            cp0.wait()
            cp1.wait()

            @pl.loop(0, CW)
            def _(r):
                @pl.loop(0, D, step=64)
                def _(col):
                    for u in range(4):
                        slc = (pl.ds(r, 1), pl.ds(col + u * 16, 16))
                        buf0.at[slc[0], slc[1]][...] = (
                            buf0.at[slc[0], slc[1]][...]
                            + buf1.at[slc[0], slc[1]][...]
                        )

            pltpu.sync_copy(buf0, o_hbm.at[pl.ds(base + c, CW)])

    return k(ys, p0, p1)


def kernel(x, expert_ids, expert_weights, gate_weights, up_weights, down_weights):
    w_pad, te, nv, p0, p1, slot, first, nxte = _dispatch_meta(
        expert_ids, expert_weights
    )
    xs = _sc_dispatch(x, p0, p1)
    ys = _tc_grouped_mlp(
        xs, w_pad, te, nv, gate_weights, up_weights, down_weights,
        slot, first, nxte,
    )
    return _sc_combine(ys, p0, p1)


# speedup vs baseline: 1.4002x; 1.0148x over previous
import functools

import jax
import jax.numpy as jnp
from jax.experimental import pallas as pl
from jax.experimental.pallas import tpu as pltpu
from jax.experimental.pallas import tpu_sc as plsc

TOKENS = 2048
D = 1024
F = 2048
E = 8
K = 2
TILE = 128
NT = (TOKENS * K) // TILE + E
R_PAD = NT * TILE
GW = 32
CW = 32

@functools.lru_cache(maxsize=1)
def _sc_mesh():
    return plsc.VectorSubcoreMesh(core_axis_name="c", subcore_axis_name="s")


def _dispatch_meta(expert_ids, expert_weights):
    N = TOKENS * K
    C = 128
    NCH = N // C

    flat_e = expert_ids.reshape(-1).astype(jnp.int32)
    flat_w = expert_weights.reshape(-1).astype(jnp.float32)

    ohf = (flat_e[:, None] == jnp.arange(E, dtype=jnp.int32)[None, :]).astype(
        jnp.float32
    )
    ohc = ohf.astype(jnp.bfloat16).reshape(NCH, C, E)
    tri = jnp.tril(jnp.ones((C, C), jnp.bfloat16))
    within = jnp.einsum(
        "rc,bce->bre", tri, ohc, preferred_element_type=jnp.float32
    )
    chunk_tot = within[:, C - 1, :]
    trix = jnp.tril(jnp.ones((NCH, NCH), jnp.bfloat16), k=-1)
    chunk_base = jnp.einsum(
        "bc,ce->be",
        trix,
        chunk_tot.astype(jnp.bfloat16),
        preferred_element_type=jnp.float32,
    )
    incl = (within + chunk_base[:, None, :]).reshape(N, E)
    rank = jnp.sum((incl - 1.0) * ohf, axis=1)

    counts = jnp.sum(chunk_tot, axis=0).astype(jnp.int32)
    tiles_per_e = (counts + TILE - 1) // TILE
    cum_tiles = jnp.cumsum(tiles_per_e)
    row_base = (cum_tiles - tiles_per_e) * TILE
    pos = (
        jnp.sum(row_base.astype(jnp.float32)[None, :] * ohf, axis=1) + rank
    ).astype(jnp.int32)

    w_pad = jnp.zeros((R_PAD,), jnp.float32).at[pos].set(flat_w)

    nvalid = cum_tiles[E - 1].astype(jnp.int32)
    tr = jnp.arange(NT, dtype=jnp.int32)
    te_raw = jnp.sum(
        (tr[:, None] >= cum_tiles[None, :]).astype(jnp.int32), axis=1
    )
    last_e = jnp.sum((cum_tiles <= nvalid - 1).astype(jnp.int32))
    te = jnp.where(tr < nvalid, jnp.minimum(te_raw, E - 1), last_e)

    p0 = pos[0::2].astype(jnp.int32)
    p1 = pos[1::2].astype(jnp.int32)

    prev = jnp.concatenate([jnp.full((1,), -1, jnp.int32), te[:-1]])
    chg = (te != prev).astype(jnp.int32)
    first = jnp.where(tr < nvalid, chg, 0).astype(jnp.int32)
    slot = ((jnp.cumsum(chg) - 1) % 2).astype(jnp.int32)
    er = jnp.arange(E, dtype=jnp.int32)
    cand = jnp.where(
        (er[None, :] > er[:, None]) & (counts[None, :] > 0), er[None, :], E
    )
    npx = jnp.min(cand, axis=1)
    npx = jnp.where(npx == E, -1, npx)
    nxte = jnp.sum(
        jnp.where(te[:, None] == er[None, :], npx[None, :], 0), axis=1
    ).astype(jnp.int32)
    return w_pad, te, nvalid.reshape(1), p0, p1, slot, first, nxte


def _sc_dispatch(x, p0, p1):
    NW = 32
    t_per_w = TOKENS // NW

    @functools.partial(
        pl.kernel,
        out_type=jax.ShapeDtypeStruct((R_PAD, D), jnp.float32),
        mesh=_sc_mesh(),
        scratch_types=[
            pltpu.VMEM((t_per_w,), jnp.int32),
            pltpu.VMEM((t_per_w,), jnp.int32),
            pltpu.VMEM((t_per_w, D), jnp.float32),
            pltpu.SemaphoreType.DMA,
            pltpu.SemaphoreType.DMA,
        ],
    )
    def k(x_hbm, p0_hbm, p1_hbm, o_hbm, i0_v, i1_v, xrows, sem0, sem1):
        wid = jax.lax.axis_index("s") * 2 + jax.lax.axis_index("c")
        base = wid * t_per_w
        pltpu.sync_copy(p0_hbm.at[pl.ds(base, t_per_w)], i0_v)
        pltpu.sync_copy(p1_hbm.at[pl.ds(base, t_per_w)], i1_v)
        pltpu.sync_copy(x_hbm.at[pl.ds(base, t_per_w)], xrows)
        c0 = pltpu.async_copy(xrows, o_hbm.at[i0_v], sem0)
        c1 = pltpu.async_copy(xrows, o_hbm.at[i1_v], sem1)
        c0.wait()
        c1.wait()

    return k(x, p0, p1)


def _mlp_body(
    te_ref,
    nv_ref,
    slot_ref,
    first_ref,
    nxte_ref,
    xs_ref,
    ws_ref,
    g_hbm,
    u_hbm,
    d_hbm,
    ys_ref,
    gbuf,
    ubuf,
    dbuf,
    gsem,
    usem,
    dsem,
):
    i = pl.program_id(0)

    def fetch(e, s):
        pltpu.make_async_copy(g_hbm.at[e], gbuf.at[s], gsem.at[s]).start()
        pltpu.make_async_copy(u_hbm.at[e], ubuf.at[s], usem.at[s]).start()
        pltpu.make_async_copy(d_hbm.at[e], dbuf.at[s], dsem.at[s]).start()

    def wait(e, s):
        pltpu.make_async_copy(g_hbm.at[e], gbuf.at[s], gsem.at[s]).wait()
        pltpu.make_async_copy(u_hbm.at[e], ubuf.at[s], usem.at[s]).wait()
        pltpu.make_async_copy(d_hbm.at[e], dbuf.at[s], dsem.at[s]).wait()

    @pl.when(i == 0)
    def _():
        fetch(te_ref[0], 0)
        nxt0 = nxte_ref[0]

        @pl.when(nxt0 >= 0)
        def _():
            fetch(nxt0, 1)

    s = slot_ref[i]

    @pl.when(jnp.logical_and(first_ref[i] == 1, i < nv_ref[0]))
    def _():
        wait(te_ref[i], s)
        nxt = nxte_ref[i]

        @pl.when(jnp.logical_and(nxt >= 0, i > 0))
        def _():
            fetch(nxt, 1 - s)

    @pl.when(i < nv_ref[0])
    def _():
        xb = xs_ref[...].astype(jnp.bfloat16)
        gw = gbuf[s].astype(jnp.bfloat16)
        uw = ubuf[s].astype(jnp.bfloat16)
        dw = dbuf[s].astype(jnp.bfloat16)
        g = jnp.dot(xb, gw, preferred_element_type=jnp.float32)
        u = jnp.dot(xb, uw, preferred_element_type=jnp.float32)
        h = (g * jax.nn.sigmoid(g) * u).astype(jnp.bfloat16)
        o = jnp.dot(h, dw, preferred_element_type=jnp.float32)
        ys_ref[...] = o * ws_ref[...]


def _tc_grouped_mlp(xs, w_pad, te, nv, gw, uw, dw, slot, first, nxte):
    grid_spec = pltpu.PrefetchScalarGridSpec(
        num_scalar_prefetch=5,
        grid=(NT,),
        in_specs=[
            pl.BlockSpec((TILE, D), lambda i, *_: (i, 0)),
            pl.BlockSpec((TILE, 1), lambda i, *_: (i, 0)),
            pl.BlockSpec(memory_space=pl.ANY),
            pl.BlockSpec(memory_space=pl.ANY),
            pl.BlockSpec(memory_space=pl.ANY),
        ],
        out_specs=pl.BlockSpec((TILE, D), lambda i, *_: (i, 0)),
        scratch_shapes=[
            pltpu.VMEM((2, D, F), jnp.float32),
            pltpu.VMEM((2, D, F), jnp.float32),
            pltpu.VMEM((2, F, D), jnp.float32),
            pltpu.SemaphoreType.DMA((2,)),
            pltpu.SemaphoreType.DMA((2,)),
            pltpu.SemaphoreType.DMA((2,)),
        ],
    )
    return pl.pallas_call(
        _mlp_body,
        grid_spec=grid_spec,
        out_shape=jax.ShapeDtypeStruct((R_PAD, D), jnp.float32),
    )(te, nv, slot, first, nxte, xs, w_pad.reshape(R_PAD, 1), gw, uw, dw)


def _sc_combine(ys, p0, p1):
    NW = 32
    t_per_w = TOKENS // NW

    @functools.partial(
        pl.kernel,
        out_type=jax.ShapeDtypeStruct((TOKENS, D), jnp.float32),
        mesh=_sc_mesh(),
        scratch_types=[
            pltpu.VMEM((t_per_w,), jnp.int32),
            pltpu.VMEM((t_per_w,), jnp.int32),
            pltpu.VMEM((CW, D), jnp.float32),
            pltpu.VMEM((CW, D), jnp.float32),
            pltpu.SemaphoreType.DMA,
            pltpu.SemaphoreType.DMA,
        ],
    )
    def k(ys_hbm, p0_hbm, p1_hbm, o_hbm, i0_v, i1_v, buf0, buf1, sem0, sem1):
        wid = jax.lax.axis_index("s") * 2 + jax.lax.axis_index("c")
        base = wid * t_per_w
        pltpu.sync_copy(p0_hbm.at[pl.ds(base, t_per_w)], i0_v)
        pltpu.sync_copy(p1_hbm.at[pl.ds(base, t_per_w)], i1_v)

        @pl.loop(0, t_per_w, step=CW)
        def _(c):
            cp0 = pltpu.async_copy(ys_hbm.at[i0_v.at[pl.ds(c, CW)]], buf0, sem0)
            cp1 = pltpu.async_copy(ys_hbm.at[i1_v.at[pl.ds(c, CW)]], buf1, sem1)
            cp0.wait()
            cp1.wait()

            @pl.loop(0, CW)
            def _(r):
                @pl.loop(0, D, step=64)
                def _(col):
                    for u in range(4):
                        slc = (pl.ds(r, 1), pl.ds(col + u * 16, 16))
                        buf0.at[slc[0], slc[1]][...] = (
                            buf0.at[slc[0], slc[1]][...]
                            + buf1.at[slc[0], slc[1]][...]
                        )

            pltpu.sync_copy(buf0, o_hbm.at[pl.ds(base + c, CW)])

    return k(ys, p0, p1)


def kernel(x, expert_ids, expert_weights, gate_weights, up_weights, down_weights):
    w_pad, te, nv, p0, p1, slot, first, nxte = _dispatch_meta(
        expert_ids, expert_weights
    )
    xs = _sc_dispatch(x, p0, p1)
    ys = _tc_grouped_mlp(
        xs, w_pad, te, nv, gate_weights, up_weights, down_weights,
        slot, first, nxte,
    )
    return _sc_combine(ys, p0, p1)
